# baseline (speedup 1.0000x reference)
import jax
import jax.numpy as jnp
from jax import lax
from jax.experimental import pallas as pl
from jax.experimental.pallas import tpu as pltpu
from jax.experimental.pallas import tpu_sc as plsc

NN = 10000
NP = 10240
NE = 320000
D = 128
LD = 64
NC = 2
NS = 16
NW = NC * NS
EPT = NE // NW
K = 80
NB = EPT // K
RPT = NP // NS

BR = 1024
GRID = NP // BR


NBUF = 5
AHEAD = 3


def _deg_body(dst_hbm, zeros1_hbm, deg_out, didx, ones_v, degbuf, deg_sp,
              *sems):
    c = lax.axis_index("c")
    s = lax.axis_index("s")
    for i in range(K // 16):
        ones_v[pl.ds(i * 16, 16)] = jnp.ones((16,), jnp.float32)
    pltpu.sync_copy(dst_hbm.at[s, pl.ds(c * NB, NB)], didx)
    pltpu.sync_copy(zeros1_hbm.at[pl.ds(s * RPT, RPT)],
                    deg_sp.at[pl.ds(s * RPT, RPT)])
    plsc.subcore_barrier()

    def loop_body(g, carry):
        for i in range(NBUF):
            b = g + i

            @pl.when(b >= NBUF)
            def _():
                pltpu.make_async_copy(ones_v, deg_sp.at[didx.at[b]],
                                      sems[i]).wait()

            pltpu.async_copy(ones_v, deg_sp.at[didx.at[b]], sems[i], add=True)
        return carry

    lax.fori_loop(0, NB // NBUF, lambda g, cy: loop_body(g * NBUF, cy), 0)
    for i in range(NBUF):
        pltpu.make_async_copy(ones_v, deg_sp.at[didx.at[i]], sems[i]).wait()
    plsc.subcore_barrier()
    pltpu.sync_copy(deg_sp.at[pl.ds(s * RPT, RPT)], degbuf)
    pltpu.sync_copy(degbuf, deg_out.at[c, pl.ds(s * RPT, RPT)])


HD = 64
EPS = NE // NS
KA = 80
NBA = EPS // KA


def _agg_body(p2_hbm, src_hbm, dst_hbm, zeros2_hbm, out_hbm,
              sidx, didx, rows, acc_sp, *sems):
    semg = sems[:NBUF]
    sems_sc = sems[NBUF:]
    c = lax.axis_index("c")
    s = lax.axis_index("s")
    pltpu.sync_copy(src_hbm.at[s], sidx)
    pltpu.sync_copy(dst_hbm.at[s], didx)
    def remap(r, carry):
        for k in range(KA // 16):
            v = sidx[r, pl.ds(k * 16, 16)]
            sidx[r, pl.ds(k * 16, 16)] = v + v + c
        return carry

    lax.fori_loop(0, NBA, remap, 0)
    pltpu.sync_copy(zeros2_hbm.at[pl.ds(s * RPT, RPT)],
                    acc_sp.at[pl.ds(s * RPT, RPT)])
    plsc.subcore_barrier()

    def gather(b, slot):
        pltpu.async_copy(p2_hbm.at[sidx.at[b]], rows.at[slot], semg[slot])

    def gather_wait(slot):
        pltpu.make_async_copy(p2_hbm.at[sidx.at[0]], rows.at[slot],
                              semg[slot]).wait()

    def scatter(b, slot):
        pltpu.async_copy(rows.at[slot], acc_sp.at[didx.at[b]],
                         sems_sc[slot], add=True)

    def scatter_wait(slot):
        pltpu.make_async_copy(rows.at[slot], acc_sp.at[didx.at[0]],
                              sems_sc[slot]).wait()

    for b in range(AHEAD):
        gather(b, b % NBUF)

    def loop_body(g, carry):
        for i in range(NBUF):
            b = g + i
            slot = i
            gather_wait(slot)
            scatter(b, slot)
            b2 = b + AHEAD
            slot2 = (i + AHEAD) % NBUF

            @pl.when(b2 < NBA)
            def _():
                @pl.when(b2 >= NBUF)
                def _():
                    scatter_wait(slot2)

                gather(b2, slot2)
        return carry

    lax.fori_loop(0, NBA // NBUF, lambda g, cy: loop_body(g * NBUF, cy), 0)
    for i in range(NBUF):
        scatter_wait(i)
    plsc.subcore_barrier()
    pltpu.sync_copy(acc_sp.at[pl.ds(s * RPT, RPT)],
                    out_hbm.at[pl.ds(s * RPT, RPT), pl.ds(c * HD, HD)])


_MESH = plsc.VectorSubcoreMesh(core_axis_name="c", subcore_axis_name="s")

_deg_call = pl.kernel(
    _deg_body,
    out_type=jax.ShapeDtypeStruct((NC, NP), jnp.float32),
    mesh=_MESH,
    scratch_types=[
        pltpu.VMEM((NB, K), jnp.int32),
        pltpu.VMEM((K,), jnp.float32),
        pltpu.VMEM((RPT,), jnp.float32),
        pltpu.VMEM_SHARED((NP,), jnp.float32),
    ] + [pltpu.SemaphoreType.DMA] * NBUF,
    compiler_params=pltpu.CompilerParams(use_tc_tiling_on_sc=False),
)

_agg_call = pl.kernel(
    _agg_body,
    out_type=jax.ShapeDtypeStruct((NP, D), jnp.float32),
    mesh=_MESH,
    scratch_types=[
        pltpu.VMEM((NBA, KA), jnp.int32),
        pltpu.VMEM((NBA, KA), jnp.int32),
        pltpu.VMEM((NBUF, KA, HD), jnp.float32),
        pltpu.VMEM_SHARED((NP, HD), jnp.float32),
    ] + [pltpu.SemaphoreType.DMA] * (2 * NBUF),
    compiler_params=pltpu.CompilerParams(use_tc_tiling_on_sc=False),
)


def _dinv(deg_ref):
    degt = jnp.transpose(deg_ref[...])
    return lax.rsqrt(degt[:, 0:1] + degt[:, 1:2] + 1.0)


def _scale_body(deg_ref, x_ref, p_ref):
    p_ref[...] = _dinv(deg_ref) * x_ref[...]


def _layer_body(deg_ref, s_ref, p_ref, w_ref, b_ref, o_ref):
    dinv = _dinv(deg_ref)
    g = dinv * (s_ref[...] + p_ref[...])
    h = jnp.tanh(
        jnp.dot(g, w_ref[...], preferred_element_type=jnp.float32,
                precision=lax.Precision.HIGHEST) + b_ref[...])
    o_ref[...] = dinv * h


def _head_body(deg_ref, s_ref, p_ref, wm_ref, bm_ref,
               wl_ref, bl_ref, noise_ref, o_ref):
    g = _dinv(deg_ref) * (s_ref[...] + p_ref[...])
    mean = jnp.dot(g, wm_ref[...], preferred_element_type=jnp.float32,
                   precision=lax.Precision.HIGHEST) + bm_ref[...]
    logvar = jnp.dot(g, wl_ref[...], preferred_element_type=jnp.float32,
                     precision=lax.Precision.HIGHEST) + bl_ref[...]
    o_ref[0] = noise_ref[...] * jnp.exp(0.5 * logvar) + mean
    o_ref[1] = mean
    o_ref[2] = logvar


def _row_spec(width):
    return pl.BlockSpec((BR, width), lambda i: (i, 0))


def _full_spec(shape):
    return pl.BlockSpec(shape, lambda i: (0,) * len(shape))


_deg_spec = pl.BlockSpec((NC, BR), lambda i: (0, i))

_scale_call = pl.pallas_call(
    _scale_body,
    out_shape=jax.ShapeDtypeStruct((NP, D), jnp.float32),
    grid=(GRID,),
    in_specs=[_deg_spec, _row_spec(D)],
    out_specs=_row_spec(D),
)

_layer_call = pl.pallas_call(
    _layer_body,
    out_shape=jax.ShapeDtypeStruct((NP, D), jnp.float32),
    grid=(GRID,),
    in_specs=[_deg_spec, _row_spec(D), _row_spec(D),
              _full_spec((D, D)), _full_spec((1, D))],
    out_specs=_row_spec(D),
)

_head_call = pl.pallas_call(
    _head_body,
    out_shape=jax.ShapeDtypeStruct((3, NN, LD), jnp.float32),
    grid=(GRID,),
    in_specs=[_deg_spec, _row_spec(D), _row_spec(D),
              _full_spec((D, LD)), _full_spec((1, LD)),
              _full_spec((D, LD)), _full_spec((1, LD)), _row_spec(LD)],
    out_specs=pl.BlockSpec((3, BR, LD), lambda i: (0, i, 0)),
)


def kernel(x, edge_index, W1, b1, W2, b2, Wm, bm, Wl, bl, noise):
    ei = edge_index.astype(jnp.int32)
    src4 = ei[0].reshape(NS, NBA, KA)
    dst4 = ei[1].reshape(NS, NBA, KA)
    zeros1 = jnp.zeros((NP,), jnp.float32)
    zeros2 = jnp.zeros((NP, HD), jnp.float32)

    def agg(p):
        return _agg_call(p.reshape(2 * NP, HD), src4, dst4, zeros2)

    deg2 = _deg_call(dst4, zeros1)
    p1 = _scale_call(deg2, x)
    s1 = agg(p1)
    p2 = _layer_call(deg2, s1, p1, W1, b1.reshape(1, D))
    s2 = agg(p2)
    p3 = _layer_call(deg2, s2, p2, W2, b2.reshape(1, D))
    s3 = agg(p3)
    out = _head_call(deg2, s3, p3,
                     Wm, bm.reshape(1, LD),
                     Wl, bl.reshape(1, LD), noise)
    return (out[0], out[1], out[2])


# speedup vs baseline: 1.0374x; 1.0374x over previous
import jax
import jax.numpy as jnp
from jax import lax
from jax.experimental import pallas as pl
from jax.experimental.pallas import tpu as pltpu
from jax.experimental.pallas import tpu_sc as plsc

NN = 10000
NP = 10240
NE = 320000
D = 128
LD = 64
NC = 2
NS = 16
NW = NC * NS
EPT = NE // NW
K = 80
NB = EPT // K
RPT = NP // NS

BR = 1024
GRID = NP // BR


NBUF = 5
AHEAD = 3


def _deg_body(dst_hbm, zeros1_hbm, deg_out, didx, ones_v, degbuf, deg_sp,
              *sems):
    c = lax.axis_index("c")
    s = lax.axis_index("s")
    for i in range(K // 16):
        ones_v[pl.ds(i * 16, 16)] = jnp.ones((16,), jnp.float32)
    pltpu.sync_copy(dst_hbm.at[s, pl.ds(c * NB, NB)], didx)
    pltpu.sync_copy(zeros1_hbm.at[pl.ds(s * RPT, RPT)],
                    deg_sp.at[pl.ds(s * RPT, RPT)])
    plsc.subcore_barrier()

    def loop_body(g, carry):
        for i in range(NBUF):
            b = g + i

            @pl.when(b >= NBUF)
            def _():
                pltpu.make_async_copy(ones_v, deg_sp.at[didx.at[b]],
                                      sems[i]).wait()

            pltpu.async_copy(ones_v, deg_sp.at[didx.at[b]], sems[i], add=True)
        return carry

    lax.fori_loop(0, NB // NBUF, lambda g, cy: loop_body(g * NBUF, cy), 0)
    for i in range(NBUF):
        pltpu.make_async_copy(ones_v, deg_sp.at[didx.at[i]], sems[i]).wait()
    plsc.subcore_barrier()
    pltpu.sync_copy(deg_sp.at[pl.ds(s * RPT, RPT)], degbuf)
    pltpu.sync_copy(degbuf, deg_out.at[c, pl.ds(s * RPT, RPT)])


HD = 64
EPS = NE // NS
KA = 80
NBA = EPS // KA


def _agg_body(p2_hbm, src_hbm, dst_hbm, zeros2_hbm, out_hbm,
              sidx, didx, rows, acc_sp, *sems):
    semg = sems[:NBUF]
    sems_sc = sems[NBUF:]
    c = lax.axis_index("c")
    s = lax.axis_index("s")
    pltpu.async_copy(src_hbm.at[s], sidx, semg[0])
    pltpu.async_copy(dst_hbm.at[s], didx, semg[1])
    pltpu.async_copy(zeros2_hbm.at[pl.ds(s * RPT, RPT)],
                     acc_sp.at[pl.ds(s * RPT, RPT)], semg[2])
    pltpu.make_async_copy(src_hbm.at[s], sidx, semg[0]).wait()

    def remap(r, carry):
        for k in range(KA // 16):
            v = sidx[r, pl.ds(k * 16, 16)]
            sidx[r, pl.ds(k * 16, 16)] = v + v + c
        return carry

    lax.fori_loop(0, NBA, remap, 0)
    pltpu.make_async_copy(dst_hbm.at[s], didx, semg[1]).wait()
    pltpu.make_async_copy(zeros2_hbm.at[pl.ds(s * RPT, RPT)],
                          acc_sp.at[pl.ds(s * RPT, RPT)], semg[2]).wait()
    plsc.subcore_barrier()

    def gather(b, slot):
        pltpu.async_copy(p2_hbm.at[sidx.at[b]], rows.at[slot], semg[slot])

    def gather_wait(slot):
        pltpu.make_async_copy(p2_hbm.at[sidx.at[0]], rows.at[slot],
                              semg[slot]).wait()

    def scatter(b, slot):
        pltpu.async_copy(rows.at[slot], acc_sp.at[didx.at[b]],
                         sems_sc[slot], add=True)

    def scatter_wait(slot):
        pltpu.make_async_copy(rows.at[slot], acc_sp.at[didx.at[0]],
                              sems_sc[slot]).wait()

    for b in range(AHEAD):
        gather(b, b % NBUF)

    def loop_body(g, carry):
        for i in range(NBUF):
            b = g + i
            slot = i
            gather_wait(slot)
            scatter(b, slot)
            b2 = b + AHEAD
            slot2 = (i + AHEAD) % NBUF

            @pl.when(b2 < NBA)
            def _():
                @pl.when(b2 >= NBUF)
                def _():
                    scatter_wait(slot2)

                gather(b2, slot2)
        return carry

    lax.fori_loop(0, NBA // NBUF, lambda g, cy: loop_body(g * NBUF, cy), 0)
    for i in range(NBUF):
        scatter_wait(i)
    plsc.subcore_barrier()
    pltpu.sync_copy(acc_sp.at[pl.ds(s * RPT, RPT)],
                    out_hbm.at[pl.ds(s * RPT, RPT), pl.ds(c * HD, HD)])


_MESH = plsc.VectorSubcoreMesh(core_axis_name="c", subcore_axis_name="s")

_deg_call = pl.kernel(
    _deg_body,
    out_type=jax.ShapeDtypeStruct((NC, NP), jnp.float32),
    mesh=_MESH,
    scratch_types=[
        pltpu.VMEM((NB, K), jnp.int32),
        pltpu.VMEM((K,), jnp.float32),
        pltpu.VMEM((RPT,), jnp.float32),
        pltpu.VMEM_SHARED((NP,), jnp.float32),
    ] + [pltpu.SemaphoreType.DMA] * NBUF,
    compiler_params=pltpu.CompilerParams(use_tc_tiling_on_sc=False),
)

_agg_call = pl.kernel(
    _agg_body,
    out_type=jax.ShapeDtypeStruct((NP, D), jnp.float32),
    mesh=_MESH,
    scratch_types=[
        pltpu.VMEM((NBA, KA), jnp.int32),
        pltpu.VMEM((NBA, KA), jnp.int32),
        pltpu.VMEM((NBUF, KA, HD), jnp.float32),
        pltpu.VMEM_SHARED((NP, HD), jnp.float32),
    ] + [pltpu.SemaphoreType.DMA] * (2 * NBUF),
    compiler_params=pltpu.CompilerParams(use_tc_tiling_on_sc=False),
)


def _dinv(deg_ref):
    degt = jnp.transpose(deg_ref[...])
    return lax.rsqrt(degt[:, 0:1] + degt[:, 1:2] + 1.0)


def _scale_body(deg_ref, x_ref, p_ref):
    p_ref[...] = _dinv(deg_ref) * x_ref[...]


def _layer_body(deg_ref, s_ref, p_ref, w_ref, b_ref, o_ref):
    dinv = _dinv(deg_ref)
    g = dinv * (s_ref[...] + p_ref[...])
    h = jnp.tanh(
        jnp.dot(g, w_ref[...], preferred_element_type=jnp.float32,
                precision=lax.Precision.HIGHEST) + b_ref[...])
    o_ref[...] = dinv * h


def _head_body(deg_ref, s_ref, p_ref, wm_ref, bm_ref,
               wl_ref, bl_ref, noise_ref, z_ref, mean_ref, logvar_ref):
    g = _dinv(deg_ref) * (s_ref[...] + p_ref[...])
    mean = jnp.dot(g, wm_ref[...], preferred_element_type=jnp.float32,
                   precision=lax.Precision.HIGHEST) + bm_ref[...]
    logvar = jnp.dot(g, wl_ref[...], preferred_element_type=jnp.float32,
                     precision=lax.Precision.HIGHEST) + bl_ref[...]
    mean_ref[...] = mean
    logvar_ref[...] = logvar
    z_ref[...] = noise_ref[...] * jnp.exp(0.5 * logvar) + mean


def _row_spec(width):
    return pl.BlockSpec((BR, width), lambda i: (i, 0))


def _full_spec(shape):
    return pl.BlockSpec(shape, lambda i: (0,) * len(shape))


_deg_spec = pl.BlockSpec((NC, BR), lambda i: (0, i))

_scale_call = pl.pallas_call(
    _scale_body,
    out_shape=jax.ShapeDtypeStruct((NP, D), jnp.float32),
    grid=(GRID,),
    in_specs=[_deg_spec, _row_spec(D)],
    out_specs=_row_spec(D),
)

_layer_call = pl.pallas_call(
    _layer_body,
    out_shape=jax.ShapeDtypeStruct((NP, D), jnp.float32),
    grid=(GRID,),
    in_specs=[_deg_spec, _row_spec(D), _row_spec(D),
              _full_spec((D, D)), _full_spec((1, D))],
    out_specs=_row_spec(D),
)

_head_call = pl.pallas_call(
    _head_body,
    out_shape=(jax.ShapeDtypeStruct((NN, LD), jnp.float32),
               jax.ShapeDtypeStruct((NN, LD), jnp.float32),
               jax.ShapeDtypeStruct((NN, LD), jnp.float32)),
    grid=(GRID,),
    in_specs=[_deg_spec, _row_spec(D), _row_spec(D),
              _full_spec((D, LD)), _full_spec((1, LD)),
              _full_spec((D, LD)), _full_spec((1, LD)), _row_spec(LD)],
    out_specs=(_row_spec(LD), _row_spec(LD), _row_spec(LD)),
)


def kernel(x, edge_index, W1, b1, W2, b2, Wm, bm, Wl, bl, noise):
    ei = edge_index.astype(jnp.int32)
    src4 = ei[0].reshape(NS, NBA, KA)
    dst4 = ei[1].reshape(NS, NBA, KA)
    zeros1 = jnp.zeros((NP,), jnp.float32)
    zeros2 = jnp.zeros((NP, HD), jnp.float32)

    def agg(p):
        return _agg_call(p.reshape(2 * NP, HD), src4, dst4, zeros2)

    deg2 = _deg_call(dst4, zeros1)
    p1 = _scale_call(deg2, x)
    s1 = agg(p1)
    p2 = _layer_call(deg2, s1, p1, W1, b1.reshape(1, D))
    s2 = agg(p2)
    p3 = _layer_call(deg2, s2, p2, W2, b2.reshape(1, D))
    s3 = agg(p3)
    z, mean, logvar = _head_call(deg2, s3, p3,
                                 Wm, bm.reshape(1, LD),
                                 Wl, bl.reshape(1, LD), noise)
    return (z, mean, logvar)


# speedup vs baseline: 1.0617x; 1.0235x over previous
import jax
import jax.numpy as jnp
from jax import lax
from jax.experimental import pallas as pl
from jax.experimental.pallas import tpu as pltpu
from jax.experimental.pallas import tpu_sc as plsc

NN = 10000
NP = 10240
NE = 320000
D = 128
LD = 64
NC = 2
NS = 16
NW = NC * NS
EPT = NE // NW
K = 80
NB = EPT // K
RPT = NP // NS

BR = 1024
GRID = NP // BR


NBUF = 5
AHEAD = 3


def _deg_body(dst_hbm, zeros1_hbm, deg_out, didx, ones_v, degbuf, deg_sp,
              *sems):
    c = lax.axis_index("c")
    s = lax.axis_index("s")
    for i in range(K // 16):
        ones_v[pl.ds(i * 16, 16)] = jnp.ones((16,), jnp.float32)
    pltpu.sync_copy(dst_hbm.at[s, pl.ds(c * NB, NB)], didx)
    pltpu.sync_copy(zeros1_hbm.at[pl.ds(s * RPT, RPT)],
                    deg_sp.at[pl.ds(s * RPT, RPT)])
    plsc.subcore_barrier()

    def loop_body(g, carry):
        for i in range(NBUF):
            b = g + i

            @pl.when(b >= NBUF)
            def _():
                pltpu.make_async_copy(ones_v, deg_sp.at[didx.at[b]],
                                      sems[i]).wait()

            pltpu.async_copy(ones_v, deg_sp.at[didx.at[b]], sems[i], add=True)
        return carry

    lax.fori_loop(0, NB // NBUF, lambda g, cy: loop_body(g * NBUF, cy), 0)
    for i in range(NBUF):
        pltpu.make_async_copy(ones_v, deg_sp.at[didx.at[i]], sems[i]).wait()
    plsc.subcore_barrier()
    pltpu.sync_copy(deg_sp.at[pl.ds(s * RPT, RPT)], degbuf)
    pltpu.sync_copy(degbuf, deg_out.at[c, pl.ds(s * RPT, RPT)])


HD = 64
EPS = NE // NS
KA = 80
NBA = EPS // KA


def _agg_body(p2_hbm, src_hbm, dst_hbm, zeros2_hbm, out_hbm,
              sidx, didx, rows, acc_sp, *sems):
    semg = sems[:NBUF]
    sems_sc = sems[NBUF:]
    c = lax.axis_index("c")
    s = lax.axis_index("s")
    pltpu.async_copy(src_hbm.at[s], sidx, semg[0])
    pltpu.async_copy(dst_hbm.at[s], didx, semg[1])
    pltpu.async_copy(zeros2_hbm.at[pl.ds(s * RPT, RPT)],
                     acc_sp.at[pl.ds(s * RPT, RPT)], semg[2])
    pltpu.make_async_copy(src_hbm.at[s], sidx, semg[0]).wait()

    def remap(r, carry):
        for k in range(KA // 16):
            v = sidx[r, pl.ds(k * 16, 16)]
            sidx[r, pl.ds(k * 16, 16)] = v + v + c
        return carry

    lax.fori_loop(0, NBA, remap, 0)
    pltpu.make_async_copy(dst_hbm.at[s], didx, semg[1]).wait()
    pltpu.make_async_copy(zeros2_hbm.at[pl.ds(s * RPT, RPT)],
                          acc_sp.at[pl.ds(s * RPT, RPT)], semg[2]).wait()
    plsc.subcore_barrier()

    def gather(b, slot):
        pltpu.async_copy(p2_hbm.at[sidx.at[b]], rows.at[slot], semg[slot])

    def gather_wait(slot):
        pltpu.make_async_copy(p2_hbm.at[sidx.at[0]], rows.at[slot],
                              semg[slot]).wait()

    def scatter(b, slot):
        pltpu.async_copy(rows.at[slot], acc_sp.at[didx.at[b]],
                         sems_sc[slot], add=True)

    def scatter_wait(slot):
        pltpu.make_async_copy(rows.at[slot], acc_sp.at[didx.at[0]],
                              sems_sc[slot]).wait()

    for b in range(AHEAD):
        gather(b, b % NBUF)

    def loop_body(g, carry):
        for i in range(NBUF):
            b = g + i
            slot = i
            gather_wait(slot)
            scatter(b, slot)
            b2 = b + AHEAD
            slot2 = (i + AHEAD) % NBUF

            @pl.when(b2 < NBA)
            def _():
                @pl.when(b2 >= NBUF)
                def _():
                    scatter_wait(slot2)

                gather(b2, slot2)
        return carry

    lax.fori_loop(0, NBA // NBUF, lambda g, cy: loop_body(g * NBUF, cy), 0)
    for i in range(NBUF):
        scatter_wait(i)
    plsc.subcore_barrier()
    pltpu.sync_copy(acc_sp.at[pl.ds(s * RPT, RPT)],
                    out_hbm.at[pl.ds(s * RPT, RPT), pl.ds(c * HD, HD)])


_MESH = plsc.VectorSubcoreMesh(core_axis_name="c", subcore_axis_name="s")

_deg_call = pl.kernel(
    _deg_body,
    out_type=jax.ShapeDtypeStruct((NC, NP), jnp.float32),
    mesh=_MESH,
    scratch_types=[
        pltpu.VMEM((NB, K), jnp.int32),
        pltpu.VMEM((K,), jnp.float32),
        pltpu.VMEM((RPT,), jnp.float32),
        pltpu.VMEM_SHARED((NP,), jnp.float32),
    ] + [pltpu.SemaphoreType.DMA] * NBUF,
    compiler_params=pltpu.CompilerParams(use_tc_tiling_on_sc=False),
)

_agg_call = pl.kernel(
    _agg_body,
    out_type=jax.ShapeDtypeStruct((NP, D), jnp.float32),
    mesh=_MESH,
    scratch_types=[
        pltpu.VMEM((NBA, KA), jnp.int32),
        pltpu.VMEM((NBA, KA), jnp.int32),
        pltpu.VMEM((NBUF, KA, HD), jnp.float32),
        pltpu.VMEM_SHARED((NP, HD), jnp.float32),
    ] + [pltpu.SemaphoreType.DMA] * (2 * NBUF),
    compiler_params=pltpu.CompilerParams(use_tc_tiling_on_sc=False),
)


def _dinv(deg_ref):
    degt = jnp.transpose(deg_ref[...])
    return lax.rsqrt(degt[:, 0:1] + degt[:, 1:2] + 1.0)


def _scale_body(deg_ref, x_ref, p_ref):
    p_ref[...] = _dinv(deg_ref) * x_ref[...]


def _layer_body(deg_ref, s_ref, p_ref, w_ref, b_ref, o_ref):
    dinv = _dinv(deg_ref)
    g = dinv * (s_ref[...] + p_ref[...])
    h = jnp.tanh(
        jnp.dot(g, w_ref[...], preferred_element_type=jnp.float32,
                precision=lax.Precision.DEFAULT) + b_ref[...])
    o_ref[...] = dinv * h


def _head_body(deg_ref, s_ref, p_ref, wm_ref, bm_ref,
               wl_ref, bl_ref, noise_ref, z_ref, mean_ref, logvar_ref):
    g = _dinv(deg_ref) * (s_ref[...] + p_ref[...])
    mean = jnp.dot(g, wm_ref[...], preferred_element_type=jnp.float32,
                   precision=lax.Precision.DEFAULT) + bm_ref[...]
    logvar = jnp.dot(g, wl_ref[...], preferred_element_type=jnp.float32,
                     precision=lax.Precision.DEFAULT) + bl_ref[...]
    mean_ref[...] = mean
    logvar_ref[...] = logvar
    z_ref[...] = noise_ref[...] * jnp.exp(0.5 * logvar) + mean


def _row_spec(width):
    return pl.BlockSpec((BR, width), lambda i: (i, 0))


def _full_spec(shape):
    return pl.BlockSpec(shape, lambda i: (0,) * len(shape))


_deg_spec = pl.BlockSpec((NC, BR), lambda i: (0, i))

_scale_call = pl.pallas_call(
    _scale_body,
    out_shape=jax.ShapeDtypeStruct((NP, D), jnp.float32),
    grid=(GRID,),
    in_specs=[_deg_spec, _row_spec(D)],
    out_specs=_row_spec(D),
)

_layer_call = pl.pallas_call(
    _layer_body,
    out_shape=jax.ShapeDtypeStruct((NP, D), jnp.float32),
    grid=(GRID,),
    in_specs=[_deg_spec, _row_spec(D), _row_spec(D),
              _full_spec((D, D)), _full_spec((1, D))],
    out_specs=_row_spec(D),
)

_head_call = pl.pallas_call(
    _head_body,
    out_shape=(jax.ShapeDtypeStruct((NN, LD), jnp.float32),
               jax.ShapeDtypeStruct((NN, LD), jnp.float32),
               jax.ShapeDtypeStruct((NN, LD), jnp.float32)),
    grid=(GRID,),
    in_specs=[_deg_spec, _row_spec(D), _row_spec(D),
              _full_spec((D, LD)), _full_spec((1, LD)),
              _full_spec((D, LD)), _full_spec((1, LD)), _row_spec(LD)],
    out_specs=(_row_spec(LD), _row_spec(LD), _row_spec(LD)),
)


def kernel(x, edge_index, W1, b1, W2, b2, Wm, bm, Wl, bl, noise):
    ei = edge_index.astype(jnp.int32)
    src4 = ei[0].reshape(NS, NBA, KA)
    dst4 = ei[1].reshape(NS, NBA, KA)
    zeros1 = jnp.zeros((NP,), jnp.float32)
    zeros2 = jnp.zeros((NP, HD), jnp.float32)

    def agg(p):
        return _agg_call(p.reshape(2 * NP, HD), src4, dst4, zeros2)

    deg2 = _deg_call(dst4, zeros1)
    p1 = _scale_call(deg2, x)
    s1 = agg(p1)
    p2 = _layer_call(deg2, s1, p1, W1, b1.reshape(1, D))
    s2 = agg(p2)
    p3 = _layer_call(deg2, s2, p2, W2, b2.reshape(1, D))
    s3 = agg(p3)
    z, mean, logvar = _head_call(deg2, s3, p3,
                                 Wm, bm.reshape(1, LD),
                                 Wl, bl.reshape(1, LD), noise)
    return (z, mean, logvar)


# speedup vs baseline: 1.0843x; 1.0213x over previous
import jax
import jax.numpy as jnp
from jax import lax
from jax.experimental import pallas as pl
from jax.experimental.pallas import tpu as pltpu
from jax.experimental.pallas import tpu_sc as plsc

NN = 10000
NP = 10240
NE = 320000
D = 128
LD = 64
NC = 2
NS = 16
NW = NC * NS
EPT = NE // NW
K = 80
NB = EPT // K
RPT = NP // NS

BR = 2048
GRID = NP // BR


NBUF = 5
AHEAD = 3


def _deg_body(dst_hbm, zeros1_hbm, deg_out, didx, ones_v, degbuf, deg_sp,
              *sems):
    c = lax.axis_index("c")
    s = lax.axis_index("s")
    for i in range(K // 16):
        ones_v[pl.ds(i * 16, 16)] = jnp.ones((16,), jnp.float32)
    pltpu.sync_copy(dst_hbm.at[s, pl.ds(c * NB, NB)], didx)
    pltpu.sync_copy(zeros1_hbm.at[pl.ds(s * RPT, RPT)],
                    deg_sp.at[pl.ds(s * RPT, RPT)])
    plsc.subcore_barrier()

    def loop_body(g, carry):
        for i in range(NBUF):
            b = g + i

            @pl.when(b >= NBUF)
            def _():
                pltpu.make_async_copy(ones_v, deg_sp.at[didx.at[b]],
                                      sems[i]).wait()

            pltpu.async_copy(ones_v, deg_sp.at[didx.at[b]], sems[i], add=True)
        return carry

    lax.fori_loop(0, NB // NBUF, lambda g, cy: loop_body(g * NBUF, cy), 0)
    for i in range(NBUF):
        pltpu.make_async_copy(ones_v, deg_sp.at[didx.at[i]], sems[i]).wait()
    plsc.subcore_barrier()
    pltpu.sync_copy(deg_sp.at[pl.ds(s * RPT, RPT)], degbuf)
    pltpu.sync_copy(degbuf, deg_out.at[c, pl.ds(s * RPT, RPT)])


HD = 64
EPS = NE // NS
KA = 80
NBA = EPS // KA


def _agg_body(p2_hbm, src_hbm, dst_hbm, zeros2_hbm, out_hbm,
              sidx, didx, rows, acc_sp, *sems):
    semg = sems[:NBUF]
    sems_sc = sems[NBUF:]
    c = lax.axis_index("c")
    s = lax.axis_index("s")
    pltpu.async_copy(src_hbm.at[s], sidx, semg[0])
    pltpu.async_copy(dst_hbm.at[s], didx, semg[1])
    pltpu.async_copy(zeros2_hbm.at[pl.ds(s * RPT, RPT)],
                     acc_sp.at[pl.ds(s * RPT, RPT)], semg[2])
    pltpu.make_async_copy(src_hbm.at[s], sidx, semg[0]).wait()

    def remap(r, carry):
        for k in range(KA // 16):
            v = sidx[r, pl.ds(k * 16, 16)]
            sidx[r, pl.ds(k * 16, 16)] = v + v + c
        return carry

    lax.fori_loop(0, NBA, remap, 0)
    pltpu.make_async_copy(dst_hbm.at[s], didx, semg[1]).wait()
    pltpu.make_async_copy(zeros2_hbm.at[pl.ds(s * RPT, RPT)],
                          acc_sp.at[pl.ds(s * RPT, RPT)], semg[2]).wait()
    plsc.subcore_barrier()

    def gather(b, slot):
        pltpu.async_copy(p2_hbm.at[sidx.at[b]], rows.at[slot], semg[slot])

    def gather_wait(slot):
        pltpu.make_async_copy(p2_hbm.at[sidx.at[0]], rows.at[slot],
                              semg[slot]).wait()

    def scatter(b, slot):
        pltpu.async_copy(rows.at[slot], acc_sp.at[didx.at[b]],
                         sems_sc[slot], add=True)

    def scatter_wait(slot):
        pltpu.make_async_copy(rows.at[slot], acc_sp.at[didx.at[0]],
                              sems_sc[slot]).wait()

    for b in range(AHEAD):
        gather(b, b % NBUF)

    def loop_body(g, carry):
        for i in range(NBUF):
            b = g + i
            slot = i
            gather_wait(slot)
            scatter(b, slot)
            b2 = b + AHEAD
            slot2 = (i + AHEAD) % NBUF

            @pl.when(b2 < NBA)
            def _():
                @pl.when(b2 >= NBUF)
                def _():
                    scatter_wait(slot2)

                gather(b2, slot2)
        return carry

    lax.fori_loop(0, NBA // NBUF, lambda g, cy: loop_body(g * NBUF, cy), 0)
    for i in range(NBUF):
        scatter_wait(i)
    plsc.subcore_barrier()
    pltpu.sync_copy(acc_sp.at[pl.ds(s * RPT, RPT)],
                    out_hbm.at[pl.ds(s * RPT, RPT), pl.ds(c * HD, HD)])


_MESH = plsc.VectorSubcoreMesh(core_axis_name="c", subcore_axis_name="s")

_deg_call = pl.kernel(
    _deg_body,
    out_type=jax.ShapeDtypeStruct((NC, NP), jnp.float32),
    mesh=_MESH,
    scratch_types=[
        pltpu.VMEM((NB, K), jnp.int32),
        pltpu.VMEM((K,), jnp.float32),
        pltpu.VMEM((RPT,), jnp.float32),
        pltpu.VMEM_SHARED((NP,), jnp.float32),
    ] + [pltpu.SemaphoreType.DMA] * NBUF,
    compiler_params=pltpu.CompilerParams(use_tc_tiling_on_sc=False),
)

_agg_call = pl.kernel(
    _agg_body,
    out_type=jax.ShapeDtypeStruct((NP, D), jnp.float32),
    mesh=_MESH,
    scratch_types=[
        pltpu.VMEM((NBA, KA), jnp.int32),
        pltpu.VMEM((NBA, KA), jnp.int32),
        pltpu.VMEM((NBUF, KA, HD), jnp.float32),
        pltpu.VMEM_SHARED((NP, HD), jnp.float32),
    ] + [pltpu.SemaphoreType.DMA] * (2 * NBUF),
    compiler_params=pltpu.CompilerParams(use_tc_tiling_on_sc=False),
)


def _dinv(deg_ref):
    degt = jnp.transpose(deg_ref[...])
    return lax.rsqrt(degt[:, 0:1] + degt[:, 1:2] + 1.0)


def _scale_body(deg_ref, x_ref, p_ref):
    p_ref[...] = _dinv(deg_ref) * x_ref[...]


def _layer_body(deg_ref, s_ref, p_ref, w_ref, b_ref, o_ref):
    dinv = _dinv(deg_ref)
    g = dinv * (s_ref[...] + p_ref[...])
    h = jnp.tanh(
        jnp.dot(g, w_ref[...], preferred_element_type=jnp.float32,
                precision=lax.Precision.DEFAULT) + b_ref[...])
    o_ref[...] = dinv * h


def _head_body(deg_ref, s_ref, p_ref, wm_ref, bm_ref,
               wl_ref, bl_ref, noise_ref, z_ref, mean_ref, logvar_ref):
    g = _dinv(deg_ref) * (s_ref[...] + p_ref[...])
    mean = jnp.dot(g, wm_ref[...], preferred_element_type=jnp.float32,
                   precision=lax.Precision.DEFAULT) + bm_ref[...]
    logvar = jnp.dot(g, wl_ref[...], preferred_element_type=jnp.float32,
                     precision=lax.Precision.DEFAULT) + bl_ref[...]
    mean_ref[...] = mean
    logvar_ref[...] = logvar
    z_ref[...] = noise_ref[...] * jnp.exp(0.5 * logvar) + mean


def _row_spec(width):
    return pl.BlockSpec((BR, width), lambda i: (i, 0))


def _full_spec(shape):
    return pl.BlockSpec(shape, lambda i: (0,) * len(shape))


_deg_spec = pl.BlockSpec((NC, BR), lambda i: (0, i))

_scale_call = pl.pallas_call(
    _scale_body,
    out_shape=jax.ShapeDtypeStruct((NP, D), jnp.float32),
    grid=(GRID,),
    in_specs=[_deg_spec, _row_spec(D)],
    out_specs=_row_spec(D),
)

_layer_call = pl.pallas_call(
    _layer_body,
    out_shape=jax.ShapeDtypeStruct((NP, D), jnp.float32),
    grid=(GRID,),
    in_specs=[_deg_spec, _row_spec(D), _row_spec(D),
              _full_spec((D, D)), _full_spec((1, D))],
    out_specs=_row_spec(D),
)

_head_call = pl.pallas_call(
    _head_body,
    out_shape=(jax.ShapeDtypeStruct((NN, LD), jnp.float32),
               jax.ShapeDtypeStruct((NN, LD), jnp.float32),
               jax.ShapeDtypeStruct((NN, LD), jnp.float32)),
    grid=(GRID,),
    in_specs=[_deg_spec, _row_spec(D), _row_spec(D),
              _full_spec((D, LD)), _full_spec((1, LD)),
              _full_spec((D, LD)), _full_spec((1, LD)), _row_spec(LD)],
    out_specs=(_row_spec(LD), _row_spec(LD), _row_spec(LD)),
)


def kernel(x, edge_index, W1, b1, W2, b2, Wm, bm, Wl, bl, noise):
    ei = edge_index.astype(jnp.int32)
    src4 = ei[0].reshape(NS, NBA, KA)
    dst4 = ei[1].reshape(NS, NBA, KA)
    zeros1 = jnp.zeros((NP,), jnp.float32)
    zeros2 = jnp.zeros((NP, HD), jnp.float32)

    def agg(p):
        return _agg_call(p.reshape(2 * NP, HD), src4, dst4, zeros2)

    deg2 = _deg_call(dst4, zeros1)
    p1 = _scale_call(deg2, x)
    s1 = agg(p1)
    p2 = _layer_call(deg2, s1, p1, W1, b1.reshape(1, D))
    s2 = agg(p2)
    p3 = _layer_call(deg2, s2, p2, W2, b2.reshape(1, D))
    s3 = agg(p3)
    z, mean, logvar = _head_call(deg2, s3, p3,
                                 Wm, bm.reshape(1, LD),
                                 Wl, bl.reshape(1, LD), noise)
    return (z, mean, logvar)
